# baseline (speedup 1.0000x reference)
import functools
import math

import jax
import jax.numpy as jnp
from jax import lax
from jax.experimental import pallas as pl
from jax.experimental.pallas import tpu as pltpu
from jax.experimental.pallas import tpu_sc as plsc

D_MODEL = 64
SCALE = math.sqrt(D_MODEL)

NC, NS, L = 2, 16, 16
NW = NC * NS

N_I = 4096
N_J = 200
LANE_BLK = 128
CPJ = 2
NCHUNK = N_J // CPJ
NBUF = 3
TPAD = 133


def _sc_embed(xq, lut):
    mesh = plsc.VectorSubcoreMesh(core_axis_name="c", subcore_axis_name="s")

    @functools.partial(
        pl.kernel,
        mesh=mesh,
        out_type=jax.ShapeDtypeStruct(
            (N_J, D_MODEL // 8, N_I // LANE_BLK, 8, LANE_BLK), jnp.float32
        ),
        scratch_types=[
            pltpu.VMEM((N_J // 8, 8 * LANE_BLK), jnp.int32),
            pltpu.VMEM((NBUF, CPJ * LANE_BLK, D_MODEL), jnp.float32),
            pltpu.VMEM((2, 8, 1, 8, TPAD), jnp.float32),
        ]
        + [pltpu.SemaphoreType.DMA] * NBUF
        + [pltpu.SemaphoreType.DMA] * 2,
        compiler_params=pltpu.CompilerParams(
            use_tc_tiling_on_sc=False, needs_layout_passes=False
        ),
    )
    def k(xq_hbm, lut_hbm, out_hbm, idx_v, g_v, t_v, *sems):
        gsem = sems[:NBUF]
        ssem = sems[NBUF:]
        c = lax.axis_index("c")
        s = lax.axis_index("s")
        wid = s * NC + c

        pltpu.sync_copy(xq_hbm.at[:, wid], idx_v)

        probe_lane = lax.iota(jnp.int32, L)

        @plsc.parallel_loop(0, (N_J // 8) * 64, unroll=4)
        def _probe_seq(t):
            jt = lax.shift_right_logical(t, 6)
            q = t & 63
            idx_v[jt, pl.ds(q * L, L)] = probe_lane + t * L

        def gather(ck, b):
            per_row = 8 // CPJ
            jt = ck // per_row
            off = lax.rem(ck, per_row) * (CPJ * LANE_BLK)
            return pltpu.make_async_copy(
                lut_hbm.at[idx_v.at[jt, pl.ds(off, CPJ * LANE_BLK)]],
                g_v.at[b],
                gsem[b],
            )

        def scatter(j, ts):
            return pltpu.make_async_copy(
                t_v.at[ts, :, :, :, pl.ds(0, LANE_BLK)],
                out_hbm.at[j, :, pl.ds(wid, 1), :, :],
                ssem[ts],
            )

        for b in range(NBUF):
            gather(b, b).start()

        lane = lax.iota(jnp.int32, L)
        zero = jnp.full((L,), 0, jnp.int32)
        dhi = [lax.shift_right_logical(lane + q * L, 3) for q in range(D_MODEL // L)]
        dlo = [(lane + q * L) & 7 for q in range(D_MODEL // L)]

        def half(j, b, half_ix, ts):
            @pl.when(j >= 2)
            def _drain(j=j, ts=ts):
                scatter(j - 2, ts).wait()

            tref = t_v.at[ts]

            @plsc.parallel_loop(0, LANE_BLK, unroll=2)
            def _transpose(r):
                rcol = zero + r
                for q in range(D_MODEL // L):
                    v = g_v[b, half_ix * LANE_BLK + r, pl.ds(q * L, L)] * SCALE
                    plsc.store_scatter(tref, [dhi[q], zero, dlo[q], rcol], v)

            scatter(j, ts).start()

        def chunk(ck, b):
            gather(ck, b).wait()
            for h in range(CPJ):
                half(ck * CPJ + h, b, h, h)

            @pl.when(ck + NBUF < NCHUNK)
            def _refill(ck=ck, b=b):
                gather(ck + NBUF, b).start()

        def outer(u, carry):
            c0 = u * NBUF
            for p in range(NBUF):
                chunk(c0 + p, p)
            return carry

        lax.fori_loop(0, NCHUNK // NBUF, outer, 0)
        for p in range(NCHUNK % NBUF):
            chunk((NCHUNK // NBUF) * NBUF + p, p)

        for j in (N_J - 2, N_J - 1):
            scatter(j, j % 2).wait()

    return k(xq, lut)


def kernel(x, lut):
    xq = (
        x.T.reshape(N_J // 8, 8, N_I // LANE_BLK, LANE_BLK)
        .transpose(0, 2, 1, 3)
        .reshape(N_J // 8, N_I // LANE_BLK, 8 * LANE_BLK)
    )
    out5 = _sc_embed(xq, lut)
    return out5.transpose(2, 4, 0, 1, 3).reshape(N_I, N_J, D_MODEL)


# speedup vs baseline: 1.0959x; 1.0959x over previous
import functools
import math

import jax
import jax.numpy as jnp
from jax import lax
from jax.experimental import pallas as pl
from jax.experimental.pallas import tpu as pltpu
from jax.experimental.pallas import tpu_sc as plsc

D_MODEL = 64
SCALE = math.sqrt(D_MODEL)

NC, NS, L = 2, 16, 16
NW = NC * NS

N_I = 4096
N_J = 200
LANE_BLK = 128
CPJ = 2
NCHUNK = N_J // CPJ
NBUF = 4
TPAD = 133


def _sc_embed(xq, lut):
    mesh = plsc.VectorSubcoreMesh(core_axis_name="c", subcore_axis_name="s")

    @functools.partial(
        pl.kernel,
        mesh=mesh,
        out_type=jax.ShapeDtypeStruct(
            (N_J, D_MODEL // 8, N_I // LANE_BLK, 8, LANE_BLK), jnp.float32
        ),
        scratch_types=[
            pltpu.VMEM((N_J // 8, 8 * LANE_BLK), jnp.int32),
            pltpu.VMEM((NBUF, CPJ * LANE_BLK, D_MODEL), jnp.float32),
            pltpu.VMEM((2, 8, 1, 8, TPAD), jnp.float32),
        ]
        + [pltpu.SemaphoreType.DMA] * NBUF
        + [pltpu.SemaphoreType.DMA] * 2,
        compiler_params=pltpu.CompilerParams(
            use_tc_tiling_on_sc=False, needs_layout_passes=False
        ),
    )
    def k(xq_hbm, lut_hbm, out_hbm, idx_v, g_v, t_v, *sems):
        gsem = sems[:NBUF]
        ssem = sems[NBUF:]
        c = lax.axis_index("c")
        s = lax.axis_index("s")
        wid = s * NC + c

        pltpu.sync_copy(xq_hbm.at[:, wid], idx_v)

        def gather(ck, b):
            per_row = 8 // CPJ
            jt = ck // per_row
            off = lax.rem(ck, per_row) * (CPJ * LANE_BLK)
            return pltpu.make_async_copy(
                lut_hbm.at[idx_v.at[jt, pl.ds(off, CPJ * LANE_BLK)]],
                g_v.at[b],
                gsem[b],
            )

        def scatter(j, ts):
            return pltpu.make_async_copy(
                t_v.at[ts, :, :, :, pl.ds(0, LANE_BLK)],
                out_hbm.at[j, :, pl.ds(wid, 1), :, :],
                ssem[ts],
            )

        for b in range(NBUF):
            gather(b, b).start()

        lane = lax.iota(jnp.int32, L)
        zero = jnp.full((L,), 0, jnp.int32)
        dhi = [lax.shift_right_logical(lane + q * L, 3) for q in range(D_MODEL // L)]
        dlo = [(lane + q * L) & 7 for q in range(D_MODEL // L)]

        def half(j, b, half_ix, ts):
            @pl.when(j >= 2)
            def _drain(j=j, ts=ts):
                scatter(j - 2, ts).wait()

            tref = t_v.at[ts]

            @plsc.parallel_loop(0, LANE_BLK, unroll=4)
            def _transpose(r):
                rcol = zero + r
                for q in range(D_MODEL // L):
                    v = g_v[b, half_ix * LANE_BLK + r, pl.ds(q * L, L)] * SCALE
                    plsc.store_scatter(tref, [dhi[q], zero, dlo[q], rcol], v)

            scatter(j, ts).start()

        def chunk(ck, b):
            gather(ck, b).wait()
            for h in range(CPJ):
                half(ck * CPJ + h, b, h, h)

            @pl.when(ck + NBUF < NCHUNK)
            def _refill(ck=ck, b=b):
                gather(ck + NBUF, b).start()

        def outer(u, carry):
            c0 = u * NBUF
            for p in range(NBUF):
                chunk(c0 + p, p)
            return carry

        lax.fori_loop(0, NCHUNK // NBUF, outer, 0)
        for p in range(NCHUNK % NBUF):
            chunk((NCHUNK // NBUF) * NBUF + p, p)

        for j in (N_J - 2, N_J - 1):
            scatter(j, j % 2).wait()

    return k(xq, lut)


def kernel(x, lut):
    xq = (
        x.T.reshape(N_J // 8, 8, N_I // LANE_BLK, LANE_BLK)
        .transpose(0, 2, 1, 3)
        .reshape(N_J // 8, N_I // LANE_BLK, 8 * LANE_BLK)
    )
    out5 = _sc_embed(xq, lut)
    return out5.transpose(2, 4, 0, 1, 3).reshape(N_I, N_J, D_MODEL)


# speedup vs baseline: 1.0994x; 1.0032x over previous
import functools
import math

import jax
import jax.numpy as jnp
from jax import lax
from jax.experimental import pallas as pl
from jax.experimental.pallas import tpu as pltpu
from jax.experimental.pallas import tpu_sc as plsc

D_MODEL = 64
SCALE = math.sqrt(D_MODEL)

NC, NS, L = 2, 16, 16
NW = NC * NS

N_I = 4096
N_J = 200
LANE_BLK = 128
CPJ = 2
NCHUNK = N_J // CPJ
NBUF = 3
TPAD = 133


def _sc_embed(xq, lut):
    mesh = plsc.VectorSubcoreMesh(core_axis_name="c", subcore_axis_name="s")

    @functools.partial(
        pl.kernel,
        mesh=mesh,
        out_type=jax.ShapeDtypeStruct(
            (N_J, D_MODEL // 8, N_I // LANE_BLK, 8, LANE_BLK), jnp.float32
        ),
        scratch_types=[
            pltpu.VMEM((N_J // 8, 8 * LANE_BLK), jnp.int32),
            pltpu.VMEM((NBUF, CPJ * LANE_BLK, D_MODEL), jnp.float32),
            pltpu.VMEM((2, 8, 1, 8, TPAD), jnp.float32),
        ]
        + [pltpu.SemaphoreType.DMA] * NBUF
        + [pltpu.SemaphoreType.DMA] * 2,
        compiler_params=pltpu.CompilerParams(
            use_tc_tiling_on_sc=False, needs_layout_passes=False
        ),
    )
    def k(xq_hbm, lut_hbm, out_hbm, idx_v, g_v, t_v, *sems):
        gsem = sems[:NBUF]
        ssem = sems[NBUF:]
        c = lax.axis_index("c")
        s = lax.axis_index("s")
        wid = s * NC + c

        pltpu.sync_copy(xq_hbm.at[:, wid], idx_v)

        def gather(ck, b):
            per_row = 8 // CPJ
            jt = ck // per_row
            off = lax.rem(ck, per_row) * (CPJ * LANE_BLK)
            return pltpu.make_async_copy(
                lut_hbm.at[idx_v.at[jt, pl.ds(off, CPJ * LANE_BLK)]],
                g_v.at[b],
                gsem[b],
            )

        def scatter(j, ts):
            return pltpu.make_async_copy(
                t_v.at[ts, :, :, :, pl.ds(0, LANE_BLK)],
                out_hbm.at[j, :, pl.ds(wid, 1), :, :],
                ssem[ts],
            )

        for b in range(NBUF):
            gather(b, b).start()

        lane = lax.iota(jnp.int32, L)
        zero = jnp.full((L,), 0, jnp.int32)
        dhi = [lax.shift_right_logical(lane + q * L, 3) for q in range(D_MODEL // L)]
        dlo = [(lane + q * L) & 7 for q in range(D_MODEL // L)]

        def half(j, b, half_ix, ts):
            @pl.when(j >= 2)
            def _drain(j=j, ts=ts):
                scatter(j - 2, ts).wait()

            tref = t_v.at[ts]

            @plsc.parallel_loop(0, LANE_BLK, unroll=2)
            def _transpose(r):
                rcol = zero + r
                for q in range(D_MODEL // L):
                    v = g_v[b, half_ix * LANE_BLK + r, pl.ds(q * L, L)] * SCALE
                    plsc.store_scatter(tref, [dhi[q], zero, dlo[q], rcol], v)

            scatter(j, ts).start()

        def chunk(ck, b):
            gather(ck, b).wait()
            for h in range(CPJ):
                half(ck * CPJ + h, b, h, h)

            @pl.when(ck + NBUF < NCHUNK)
            def _refill(ck=ck, b=b):
                gather(ck + NBUF, b).start()

        def outer(u, carry):
            c0 = u * NBUF
            for p in range(NBUF):
                chunk(c0 + p, p)
            return carry

        lax.fori_loop(0, NCHUNK // NBUF, outer, 0)
        for p in range(NCHUNK % NBUF):
            chunk((NCHUNK // NBUF) * NBUF + p, p)

        for j in (N_J - 2, N_J - 1):
            scatter(j, j % 2).wait()

    return k(xq, lut)


def kernel(x, lut):
    xq = (
        x.T.reshape(N_J // 8, 8, N_I // LANE_BLK, LANE_BLK)
        .transpose(0, 2, 1, 3)
        .reshape(N_J // 8, N_I // LANE_BLK, 8 * LANE_BLK)
    )
    out5 = _sc_embed(xq, lut)
    return out5.transpose(2, 4, 0, 1, 3).reshape(N_I, N_J, D_MODEL)
